# baseline (speedup 1.0000x reference)
import jax
import jax.numpy as jnp
from jax.experimental import pallas as pl

N_SWARM = 64
K_ACTIVE = 8
N_BASIS = 256
BLOCK_T = 4096
GAP_THRESH = 2.5e-4
EXACT_CAP = 4096

_SIN_Q = (0.1013211831053168, -0.006620879676771933, 0.00017350554234663357,
          -2.522503655172663e-06, 2.3349986675745213e-08,
          -1.3452499184052336e-10)
_TP_HI = 6.28125
_TP_LO = 0.0019353071693331003
_INV_2PI = 0.15915493667125702
_PISQ = 9.869604401089358


def _poly_body(x_ref, f2_ref, p2_ref, amps_ref, biases_ref, gw_ref,
               fs_ref, fb_ref, out_ref, flag_ref):
    xb = x_ref[...]
    a = f2_ref[...] * xb + p2_ref[...]
    m = jnp.round(a * jnp.float32(_INV_2PI))
    th = (a - m * jnp.float32(_TP_HI)) - m * jnp.float32(_TP_LO)
    t = th * th
    q = jnp.float32(_SIN_Q[5])
    for c in _SIN_Q[4::-1]:
        q = q * t + jnp.float32(c)
    basis = (th * (jnp.float32(_PISQ) - t)) * q

    basis_sum = jnp.sum(basis, axis=0, keepdims=True)
    logits = jnp.dot(gw_ref[...], basis,
                     preferred_element_type=jnp.float32)

    m1 = jnp.max(logits, axis=0, keepdims=True)
    L = logits
    m = m1
    for _ in range(K_ACTIVE - 1):
        L = jnp.where(L == m, -jnp.inf, L)
        m = jnp.max(L, axis=0, keepdims=True)
    L = jnp.where(L == m, -jnp.inf, L)
    m9 = jnp.max(L, axis=0, keepdims=True)
    sel = (logits >= m)
    n_sel = jnp.sum(sel.astype(jnp.float32), axis=0, keepdims=True)
    needs_exact = jnp.logical_or(m - m9 < GAP_THRESH,
                                 n_sel != jnp.float32(K_ACTIVE))
    flag_ref[...] = needs_exact.astype(jnp.float32)

    w = jnp.where(sel, jnp.exp(logits - m1), 0.0)
    sum_w = jnp.sum(w, axis=0, keepdims=True)
    sum_wa = jnp.sum(w * amps_ref[...], axis=0, keepdims=True)
    sum_wb = jnp.sum(w * biases_ref[...], axis=0, keepdims=True)
    out = (basis_sum * sum_wa + sum_wb) / sum_w
    out_ref[...] = fs_ref[0, 0] * out + fb_ref[0, 0]


def _exact_body(x_ref, f_ref, p_ref, amps_ref, biases_ref, gw_ref,
                fs_ref, fb_ref, out_ref):
    xb = x_ref[...]
    basis = jnp.sin(f_ref[...] * xb + p_ref[...])
    basis_sum = jnp.sum(basis, axis=0, keepdims=True)
    logits = jnp.dot(gw_ref[...], basis,
                     preferred_element_type=jnp.float32)

    expert = jax.lax.broadcasted_iota(jnp.int32, logits.shape, 0)
    m1 = jnp.max(logits, axis=0, keepdims=True)
    L = logits
    mask = jnp.zeros(logits.shape, dtype=jnp.bool_)
    m = m1
    for k in range(K_ACTIVE):
        m = m1 if k == 0 else jnp.max(L, axis=0, keepdims=True)
        sel = jnp.min(jnp.where(L == m, expert, N_SWARM),
                      axis=0, keepdims=True)
        onehot = expert == sel
        mask = jnp.logical_or(mask, onehot)
        L = jnp.where(onehot, -jnp.inf, L)

    w = jnp.where(mask, jnp.exp(logits - m1), 0.0)
    sum_w = jnp.sum(w, axis=0, keepdims=True)
    sum_wa = jnp.sum(w * amps_ref[...], axis=0, keepdims=True)
    sum_wb = jnp.sum(w * biases_ref[...], axis=0, keepdims=True)
    out = (basis_sum * sum_wa + sum_wb) / sum_w
    out_ref[...] = fs_ref[0, 0] * out + fb_ref[0, 0]


def _rep_spec(shape):
    return pl.BlockSpec(shape, lambda i: (0, 0))


@jax.jit
def kernel(x, freqs, phases, amps, biases, gate_w, final_scale, final_bias):
    B = x.shape[0]
    xr = x.reshape(1, B)
    fcol = freqs.reshape(N_BASIS, 1)
    pcol = phases.reshape(N_BASIS, 1)
    f2 = fcol
    p2 = pcol
    amps_c = amps.reshape(N_SWARM, 1)
    biases_c = biases.reshape(N_SWARM, 1)
    fs = final_scale.reshape(1, 1)
    fb = final_bias.reshape(1, 1)

    out_poly, flag_f = pl.pallas_call(
        _poly_body,
        grid=(B // BLOCK_T,),
        in_specs=[
            pl.BlockSpec((1, BLOCK_T), lambda i: (0, i)),
            _rep_spec((N_BASIS, 1)),
            _rep_spec((N_BASIS, 1)),
            _rep_spec((N_SWARM, 1)),
            _rep_spec((N_SWARM, 1)),
            _rep_spec((N_SWARM, N_BASIS)),
            _rep_spec((1, 1)),
            _rep_spec((1, 1)),
        ],
        out_specs=(pl.BlockSpec((1, BLOCK_T), lambda i: (0, i)),
                   pl.BlockSpec((1, BLOCK_T), lambda i: (0, i))),
        out_shape=(jax.ShapeDtypeStruct((1, B), jnp.float32),
                   jax.ShapeDtypeStruct((1, B), jnp.float32)),
    )(xr, f2, p2, amps_c, biases_c, gate_w, fs, fb)

    flag = (flag_f.reshape(B) > 0.5).astype(jnp.int32)
    pos = jnp.cumsum(flag) - 1
    count = pos[-1] + 1
    idx = jnp.full((EXACT_CAP,), B, dtype=jnp.int32)
    idx = idx.at[jnp.where(flag == 1, pos, EXACT_CAP)].set(
        jnp.arange(B, dtype=jnp.int32), mode='drop')
    xg = jnp.where(idx < B, x.reshape(B)[jnp.minimum(idx, B - 1)], 0.0)

    out_exact = pl.pallas_call(
        _exact_body,
        grid=(1,),
        in_specs=[
            pl.BlockSpec((1, EXACT_CAP), lambda i: (0, 0)),
            _rep_spec((N_BASIS, 1)),
            _rep_spec((N_BASIS, 1)),
            _rep_spec((N_SWARM, 1)),
            _rep_spec((N_SWARM, 1)),
            _rep_spec((N_SWARM, N_BASIS)),
            _rep_spec((1, 1)),
            _rep_spec((1, 1)),
        ],
        out_specs=pl.BlockSpec((1, EXACT_CAP), lambda i: (0, 0)),
        out_shape=jax.ShapeDtypeStruct((1, EXACT_CAP), jnp.float32),
    )(xg.reshape(1, EXACT_CAP), fcol, pcol, amps_c, biases_c, gate_w, fs, fb)

    valid = jnp.arange(EXACT_CAP, dtype=jnp.int32) < count
    scatter_idx = jnp.where(valid, idx, B)
    out = out_poly.reshape(B).at[scatter_idx].set(
        out_exact.reshape(EXACT_CAP), mode='drop')
    return out.reshape(B, 1)


# speedup vs baseline: 1.5728x; 1.5728x over previous
import jax
import jax.numpy as jnp
from jax.experimental import pallas as pl

N_SWARM = 64
K_ACTIVE = 8
N_BASIS = 256
BLOCK_T = 4096


def _slsn_body(x_ref, f_ref, p_ref, amps_ref, biases_ref, gw_ref,
               fs_ref, fb_ref, out_ref):
    xb = x_ref[...]
    basis = jnp.sin(f_ref[...] * xb + p_ref[...])
    basis_sum = jnp.sum(basis, axis=0, keepdims=True)
    logits = jnp.dot(gw_ref[...], basis,
                     preferred_element_type=jnp.float32)

    expert = jax.lax.broadcasted_iota(jnp.int32, logits.shape, 0)
    m1 = jnp.max(logits, axis=0, keepdims=True)
    L = logits
    mask = jnp.zeros(logits.shape, dtype=jnp.bool_)
    m = m1
    for k in range(K_ACTIVE):
        m = m1 if k == 0 else jnp.max(L, axis=0, keepdims=True)
        sel = jnp.min(jnp.where(L == m, expert, N_SWARM),
                      axis=0, keepdims=True)
        onehot = expert == sel
        mask = jnp.logical_or(mask, onehot)
        L = jnp.where(onehot, -jnp.inf, L)

    w = jnp.where(mask, jnp.exp(logits - m1), 0.0)
    sum_w = jnp.sum(w, axis=0, keepdims=True)
    sum_wa = jnp.sum(w * amps_ref[...], axis=0, keepdims=True)
    sum_wb = jnp.sum(w * biases_ref[...], axis=0, keepdims=True)
    out = (basis_sum * sum_wa + sum_wb) / sum_w
    out_ref[...] = fs_ref[0, 0] * out + fb_ref[0, 0]


def _rep_spec(shape):
    return pl.BlockSpec(shape, lambda i: (0, 0))


@jax.jit
def kernel(x, freqs, phases, amps, biases, gate_w, final_scale, final_bias):
    B = x.shape[0]
    out = pl.pallas_call(
        _slsn_body,
        grid=(B // BLOCK_T,),
        in_specs=[
            pl.BlockSpec((1, BLOCK_T), lambda i: (0, i)),
            _rep_spec((N_BASIS, 1)),
            _rep_spec((N_BASIS, 1)),
            _rep_spec((N_SWARM, 1)),
            _rep_spec((N_SWARM, 1)),
            _rep_spec((N_SWARM, N_BASIS)),
            _rep_spec((1, 1)),
            _rep_spec((1, 1)),
        ],
        out_specs=pl.BlockSpec((1, BLOCK_T), lambda i: (0, i)),
        out_shape=jax.ShapeDtypeStruct((1, B), jnp.float32),
    )(x.reshape(1, B), freqs.reshape(N_BASIS, 1), phases.reshape(N_BASIS, 1),
      amps.reshape(N_SWARM, 1), biases.reshape(N_SWARM, 1), gate_w,
      final_scale.reshape(1, 1), final_bias.reshape(1, 1))
    return out.reshape(B, 1)
